# baseline (speedup 1.0000x reference)
import jax
import jax.numpy as jnp
from jax import lax
from jax.experimental import pallas as pl
from jax.experimental.pallas import tpu as pltpu
from jax.experimental.pallas import tpu_sc as plsc

N = 320000
D = 128
S = 10000

NH = 2
NROW = N // NH

NC = 2
NS = 16
NW = NC * NS
ROWS_PER_W = NROW // NW
SCAT = 40
NOUT = ROWS_PER_W // SCAT
NBUF = 4
NLOOP = NOUT // NBUF
TAIL = NOUT - NLOOP * NBUF
SEG_PER_TILE = 624
REM_START = SEG_PER_TILE * NS
REM = S - REM_START


def _phi_body(x_ref, w1_ref, b1_ref, w2_ref, b2_ref, h_ref):
    xb = x_ref[...].astype(jnp.bfloat16)
    u = jnp.maximum(
        jnp.dot(xb, w1_ref[...], preferred_element_type=jnp.float32)
        + b1_ref[...], 0.0)
    h_ref[...] = (
        jnp.dot(u.astype(jnp.bfloat16), w2_ref[...],
                preferred_element_type=jnp.float32)
        + b2_ref[...])


def _phi(x, W1, b1, W2, b2, block=8000):
    grid = (NROW // block,)
    return pl.pallas_call(
        _phi_body,
        grid=grid,
        in_specs=[
            pl.BlockSpec((block, D), lambda i: (i, 0)),
            pl.BlockSpec((D, D), lambda i: (0, 0)),
            pl.BlockSpec((1, D), lambda i: (0, 0)),
            pl.BlockSpec((D, D), lambda i: (0, 0)),
            pl.BlockSpec((1, D), lambda i: (0, 0)),
        ],
        out_specs=pl.BlockSpec((block, D), lambda i: (i, 0)),
        out_shape=jax.ShapeDtypeStruct((NROW, D), jnp.float32),
    )(x, W1, b1, W2, b2)


def _seg_sum_body(h_hbm, seg_hbm, zero_hbm, out_hbm, acc_shared, idx_all,
                  buf0, buf1, buf2, buf3, sem0, sem1, sem2, sem3):
    c = lax.axis_index("c")
    s = lax.axis_index("s")
    wid = c * NS + s
    base = wid * ROWS_PER_W

    pltpu.sync_copy(zero_hbm.at[pl.ds(s * SEG_PER_TILE, SEG_PER_TILE), :],
                    acc_shared.at[pl.ds(s * SEG_PER_TILE, SEG_PER_TILE), :])

    @pl.when(s == NS - 1)
    def _zero_rem():
        pltpu.sync_copy(zero_hbm.at[pl.ds(REM_START, REM), :],
                        acc_shared.at[pl.ds(REM_START, REM), :])

    pltpu.sync_copy(seg_hbm.at[wid], idx_all)

    plsc.subcore_barrier()

    bufs = (buf0, buf1, buf2, buf3)
    sems = (sem0, sem1, sem2, sem3)

    def scatter_chunk(k, buf):
        pltpu.sync_copy(buf, acc_shared.at[idx_all.at[k]], add=True)

    for b in range(NBUF):
        pltpu.async_copy(h_hbm.at[pl.ds(base + b * SCAT, SCAT), :],
                         bufs[b], sems[b])

    def body(g, _):
        for b in range(NBUF):
            k = NBUF * g + b
            pltpu.make_async_copy(h_hbm.at[pl.ds(0, SCAT), :],
                                  bufs[b], sems[b]).wait()
            scatter_chunk(k, bufs[b])

            @pl.when(k + NBUF < NOUT)
            def _prefetch():
                pltpu.async_copy(
                    h_hbm.at[pl.ds(base + (k + NBUF) * SCAT, SCAT), :],
                    bufs[b], sems[b])
        return _

    lax.fori_loop(0, NLOOP, body, None)

    for b in range(TAIL):
        k = NLOOP * NBUF + b
        pltpu.make_async_copy(h_hbm.at[pl.ds(0, SCAT), :],
                              bufs[b], sems[b]).wait()
        scatter_chunk(k, bufs[b])

    plsc.subcore_barrier()

    pltpu.sync_copy(acc_shared.at[pl.ds(s * SEG_PER_TILE, SEG_PER_TILE), :],
                    out_hbm.at[c, pl.ds(s * SEG_PER_TILE, SEG_PER_TILE), :])

    @pl.when(s == NS - 1)
    def _flush_rem():
        pltpu.sync_copy(acc_shared.at[pl.ds(REM_START, REM), :],
                        out_hbm.at[c, pl.ds(REM_START, REM), :])


def _seg_sum(h, seg2d, zero):
    mesh = plsc.VectorSubcoreMesh(core_axis_name="c", subcore_axis_name="s",
                                  num_cores=NC, num_subcores=NS)
    f = pl.kernel(
        _seg_sum_body,
        out_type=jax.ShapeDtypeStruct((NC, S, D), jnp.float32),
        mesh=mesh,
        scratch_types=[
            pltpu.VMEM_SHARED((S, D), jnp.float32),
            pltpu.VMEM((NOUT, SCAT), jnp.int32),
            pltpu.VMEM((SCAT, D), jnp.float32),
            pltpu.VMEM((SCAT, D), jnp.float32),
            pltpu.VMEM((SCAT, D), jnp.float32),
            pltpu.VMEM((SCAT, D), jnp.float32),
            pltpu.SemaphoreType.DMA,
            pltpu.SemaphoreType.DMA,
            pltpu.SemaphoreType.DMA,
            pltpu.SemaphoreType.DMA,
        ],
    )
    return f(h, seg2d, zero)


def _rho_body(p0_ref, p1_ref, w3_ref, b3_ref, w4_ref, b4_ref, out_ref):
    xs = (p0_ref[0] + p0_ref[1]) + (p1_ref[0] + p1_ref[1])
    u = jnp.maximum(
        jnp.dot(xs.astype(jnp.bfloat16), w3_ref[...],
                preferred_element_type=jnp.float32)
        + b3_ref[...], 0.0)
    out_ref[...] = (
        jnp.dot(u.astype(jnp.bfloat16), w4_ref[...],
                preferred_element_type=jnp.float32)
        + b4_ref[...])


def _rho(p0, p1, W3, b3, W4, b4, block=1000):
    grid = (S // block,)
    return pl.pallas_call(
        _rho_body,
        grid=grid,
        in_specs=[
            pl.BlockSpec((NC, block, D), lambda i: (0, i, 0)),
            pl.BlockSpec((NC, block, D), lambda i: (0, i, 0)),
            pl.BlockSpec((D, D), lambda i: (0, 0)),
            pl.BlockSpec((1, D), lambda i: (0, 0)),
            pl.BlockSpec((D, D), lambda i: (0, 0)),
            pl.BlockSpec((1, D), lambda i: (0, 0)),
        ],
        out_specs=pl.BlockSpec((block, D), lambda i: (i, 0)),
        out_shape=jax.ShapeDtypeStruct((S, D), jnp.float32),
    )(p0, p1, W3, b3, W4, b4)


def kernel(x, segment_ids, W1, b1, W2, b2, W3, b3, W4, b4):
    seg4d = segment_ids.astype(jnp.int32).reshape(NH, NW, NOUT, SCAT)
    W1b = W1.astype(jnp.bfloat16)
    W2b = W2.astype(jnp.bfloat16)
    b1r = b1.reshape(1, D)
    b2r = b2.reshape(1, D)
    zero = jnp.zeros((S, D), jnp.float32)

    h0 = _phi(x[:NROW], W1b, b1r, W2b, b2r)
    h1 = _phi(x[NROW:], W1b, b1r, W2b, b2r)
    p0 = _seg_sum(h0, seg4d[0], zero)
    p1 = _seg_sum(h1, seg4d[1], zero)
    return _rho(p0, p1, W3.astype(jnp.bfloat16), b3.reshape(1, D),
                W4.astype(jnp.bfloat16), b4.reshape(1, D))


# speedup vs baseline: 1.4408x; 1.4408x over previous
import functools

import jax
import jax.numpy as jnp
from jax import lax
from jax.experimental import pallas as pl
from jax.experimental.pallas import tpu as pltpu
from jax.experimental.pallas import tpu_sc as plsc

N = 320000
D = 128
S = 10000

NC = 2
NS = 16
NW = NC * NS
ROWS_PER_W = N // NW
SCAT = 80
NOUT = ROWS_PER_W // SCAT
NBUF = 3
NLOOP = NOUT // NBUF
TAIL = NOUT - NLOOP * NBUF
SEG_PER_TILE = 624
REM_START = SEG_PER_TILE * NS
REM = S - REM_START


def _phi_body(x_ref, w1_ref, b1_ref, w2_ref, b2_ref, h_ref):
    xb = x_ref[...].astype(jnp.bfloat16)
    u = jnp.maximum(
        jnp.dot(xb, w1_ref[...], preferred_element_type=jnp.float32)
        + b1_ref[...], 0.0)
    h_ref[...] = (
        jnp.dot(u.astype(jnp.bfloat16), w2_ref[...],
                preferred_element_type=jnp.float32)
        + b2_ref[...])


def _phi(x, W1, b1, W2, b2, block=16000):
    grid = (N // block,)
    return pl.pallas_call(
        _phi_body,
        grid=grid,
        in_specs=[
            pl.BlockSpec((block, D), lambda i: (i, 0)),
            pl.BlockSpec((D, D), lambda i: (0, 0)),
            pl.BlockSpec((1, D), lambda i: (0, 0)),
            pl.BlockSpec((D, D), lambda i: (0, 0)),
            pl.BlockSpec((1, D), lambda i: (0, 0)),
        ],
        out_specs=pl.BlockSpec((block, D), lambda i: (i, 0)),
        out_shape=jax.ShapeDtypeStruct((N, D), jnp.float32),
    )(x, W1, b1, W2, b2)


def _seg_sum_body(h_hbm, seg_hbm, zero_hbm, out_hbm, acc_shared, idx_all,
                  buf0, buf1, buf2, sem0, sem1, sem2):
    c = lax.axis_index("c")
    s = lax.axis_index("s")
    wid = c * NS + s
    base = wid * ROWS_PER_W

    pltpu.sync_copy(zero_hbm.at[pl.ds(s * SEG_PER_TILE, SEG_PER_TILE), :],
                    acc_shared.at[pl.ds(s * SEG_PER_TILE, SEG_PER_TILE), :])

    @pl.when(s == NS - 1)
    def _zero_rem():
        pltpu.sync_copy(zero_hbm.at[pl.ds(REM_START, REM), :],
                        acc_shared.at[pl.ds(REM_START, REM), :])

    pltpu.sync_copy(seg_hbm.at[wid], idx_all)

    plsc.subcore_barrier()

    bufs = (buf0, buf1, buf2)
    sems = (sem0, sem1, sem2)

    def scatter_chunk(k, buf):
        pltpu.sync_copy(buf, acc_shared.at[idx_all.at[k]], add=True)

    for b in range(NBUF):
        pltpu.async_copy(h_hbm.at[pl.ds(base + b * SCAT, SCAT), :],
                         bufs[b], sems[b])

    def body(g, _):
        for b in range(NBUF):
            k = NBUF * g + b
            pltpu.make_async_copy(h_hbm.at[pl.ds(0, SCAT), :],
                                  bufs[b], sems[b]).wait()
            scatter_chunk(k, bufs[b])

            @pl.when(k + NBUF < NOUT)
            def _prefetch():
                pltpu.async_copy(
                    h_hbm.at[pl.ds(base + (k + NBUF) * SCAT, SCAT), :],
                    bufs[b], sems[b])
        return _

    lax.fori_loop(0, NLOOP, body, None)

    for b in range(TAIL):
        k = NLOOP * NBUF + b
        pltpu.make_async_copy(h_hbm.at[pl.ds(0, SCAT), :],
                              bufs[b], sems[b]).wait()
        scatter_chunk(k, bufs[b])

    plsc.subcore_barrier()

    pltpu.sync_copy(acc_shared.at[pl.ds(s * SEG_PER_TILE, SEG_PER_TILE), :],
                    out_hbm.at[c, pl.ds(s * SEG_PER_TILE, SEG_PER_TILE), :])

    @pl.when(s == NS - 1)
    def _flush_rem():
        pltpu.sync_copy(acc_shared.at[pl.ds(REM_START, REM), :],
                        out_hbm.at[c, pl.ds(REM_START, REM), :])


def _seg_sum(h, seg2d, zero):
    mesh = plsc.VectorSubcoreMesh(core_axis_name="c", subcore_axis_name="s",
                                  num_cores=NC, num_subcores=NS)
    f = pl.kernel(
        _seg_sum_body,
        out_type=jax.ShapeDtypeStruct((NC, S, D), jnp.float32),
        mesh=mesh,
        scratch_types=[
            pltpu.VMEM_SHARED((S, D), jnp.float32),
            pltpu.VMEM((NOUT, SCAT), jnp.int32),
            pltpu.VMEM((SCAT, D), jnp.float32),
            pltpu.VMEM((SCAT, D), jnp.float32),
            pltpu.VMEM((SCAT, D), jnp.float32),
            pltpu.SemaphoreType.DMA,
            pltpu.SemaphoreType.DMA,
            pltpu.SemaphoreType.DMA,
        ],
    )
    return f(h, seg2d, zero)


def _rho_body(p_ref, w3_ref, b3_ref, w4_ref, b4_ref, out_ref):
    xs = p_ref[0].astype(jnp.float32) + p_ref[1].astype(jnp.float32)
    u = jnp.maximum(
        jnp.dot(xs.astype(jnp.bfloat16), w3_ref[...],
                preferred_element_type=jnp.float32)
        + b3_ref[...], 0.0)
    out_ref[...] = (
        jnp.dot(u.astype(jnp.bfloat16), w4_ref[...],
                preferred_element_type=jnp.float32)
        + b4_ref[...])


def _rho(partials, W3, b3, W4, b4, block=1000):
    grid = (S // block,)
    return pl.pallas_call(
        _rho_body,
        grid=grid,
        in_specs=[
            pl.BlockSpec((NC, block, D), lambda i: (0, i, 0)),
            pl.BlockSpec((D, D), lambda i: (0, 0)),
            pl.BlockSpec((1, D), lambda i: (0, 0)),
            pl.BlockSpec((D, D), lambda i: (0, 0)),
            pl.BlockSpec((1, D), lambda i: (0, 0)),
        ],
        out_specs=pl.BlockSpec((block, D), lambda i: (i, 0)),
        out_shape=jax.ShapeDtypeStruct((S, D), jnp.float32),
    )(partials, W3, b3, W4, b4)


def kernel(x, segment_ids, W1, b1, W2, b2, W3, b3, W4, b4):
    seg2d = segment_ids.astype(jnp.int32).reshape(NW, ROWS_PER_W // SCAT, SCAT)
    h = _phi(x, W1.astype(jnp.bfloat16), b1.reshape(1, D),
             W2.astype(jnp.bfloat16), b2.reshape(1, D))
    zero = jnp.zeros((S, D), jnp.float32)
    partials = _seg_sum(h, seg2d, zero)
    return _rho(partials, W3.astype(jnp.bfloat16), b3.reshape(1, D),
                W4.astype(jnp.bfloat16), b4.reshape(1, D))
